# baseline (speedup 1.0000x reference)
import functools

import jax
import jax.numpy as jnp
from jax import lax
from jax.experimental import pallas as pl
from jax.experimental.pallas import tpu as pltpu
from jax.experimental.pallas import tpu_sc as plsc

B, S, D = 4, 2048, 1024
NC, NS = 2, 16
NW = NC * NS
ROWS_PER_W = S // NW
CH = 16
NCHUNK = ROWS_PER_W // CH

_mesh = plsc.VectorSubcoreMesh(
    core_axis_name="c", subcore_axis_name="s", num_cores=NC, num_subcores=NS
)


def _sc_body(in_hbm, pos_hbm, out_hbm, pos_v, in_v):
    wid = lax.axis_index("s") * NC + lax.axis_index("c")
    base = wid * ROWS_PER_W

    def chunk_body(k, carry):
        r0 = base + k * CH
        pltpu.sync_copy(pos_hbm.at[pl.ds(r0, CH)], pos_v)
        for b in range(B):
            pltpu.sync_copy(in_hbm.at[b, pl.ds(r0, CH)], in_v)

            def row_body(r, c2):
                for cc in range(D // 16):
                    sl = pl.ds(cc * 16, 16)
                    plsc.addupdate(in_v.at[r, sl], pos_v[r, sl])
                return c2

            lax.fori_loop(0, CH, row_body, 0)
            pltpu.sync_copy(in_v, out_hbm.at[b, pl.ds(r0, CH)])
        return carry

    lax.fori_loop(0, NCHUNK, chunk_body, 0)


def _sc_add(inputs, pos_table):
    f = pl.kernel(
        _sc_body,
        out_type=jax.ShapeDtypeStruct((B, S, D), jnp.float32),
        mesh=_mesh,
        scratch_types=[
            pltpu.VMEM((CH, D), jnp.float32),
            pltpu.VMEM((CH, D), jnp.float32),
        ],
    )
    return f(inputs, pos_table)


def _tc_add_kernel(x_ref, p_ref, o_ref):
    o_ref[...] = x_ref[...] + p_ref[...]


def _tc_add(inputs, pos_table):
    b, l, d = inputs.shape
    bl = 2048
    grid = (l // bl, b)
    return pl.pallas_call(
        _tc_add_kernel,
        grid=grid,
        in_specs=[
            pl.BlockSpec((1, bl, d), lambda i, bb: (bb, i, 0)),
            pl.BlockSpec((bl, d), lambda i, bb: (i, 0)),
        ],
        out_specs=pl.BlockSpec((1, bl, d), lambda i, bb: (bb, i, 0)),
        out_shape=jax.ShapeDtypeStruct(inputs.shape, inputs.dtype),
    )(inputs, pos_table)


def kernel(inputs, pos_table):
    return _sc_add(inputs, pos_table)
